# baseline (speedup 1.0000x reference)
import functools
import math

import jax
import jax.numpy as jnp
from jax import lax
from jax.experimental import pallas as pl
from jax.experimental.pallas import tpu as pltpu
from jax.experimental.pallas import tpu_sc as plsc

D_MODEL = 64
SCALE = math.sqrt(D_MODEL)
NUM_CORES = 2
NUM_SUBCORES = 16
NUM_WORKERS = NUM_CORES * NUM_SUBCORES
LANES = 16
B_TOTAL = 4096 * 200
PER_WORKER = B_TOTAL // NUM_WORKERS
CHUNK = 512
NCHUNKS = PER_WORKER // CHUNK


def _embed(idx_flat, table):
    mesh = plsc.VectorSubcoreMesh(
        core_axis_name="c", subcore_axis_name="s",
        num_cores=NUM_CORES, num_subcores=NUM_SUBCORES)

    @functools.partial(
        pl.kernel,
        out_type=jax.ShapeDtypeStruct((B_TOTAL, D_MODEL), jnp.float32),
        mesh=mesh,
        scratch_types=[
            pltpu.VMEM((PER_WORKER,), jnp.int32),
            pltpu.VMEM((CHUNK, D_MODEL), jnp.float32),
            pltpu.VMEM((CHUNK, D_MODEL), jnp.float32),
            pltpu.SemaphoreType.DMA,
            pltpu.SemaphoreType.DMA,
        ],
        compiler_params=pltpu.CompilerParams(use_tc_tiling_on_sc=False),
    )
    def emb_kernel(idx_hbm, table_hbm, out_hbm, idx_v, rows0, rows1, s0, s1):
        wid = lax.axis_index("s") * NUM_CORES + lax.axis_index("c")
        base = wid * PER_WORKER
        pltpu.sync_copy(idx_hbm.at[pl.ds(base, PER_WORKER)], idx_v)

        bufs = (rows0, rows1)
        sems = (s0, s1)

        def gather_start(g, b):
            pltpu.async_copy(
                table_hbm.at[idx_v.at[pl.ds(g * CHUNK, CHUNK)]],
                bufs[b], sems[b])

        def scale_store(g, b):
            rows = bufs[b]

            def scale_body(i):
                r = i // (D_MODEL // LANES)
                col = (i % (D_MODEL // LANES)) * LANES
                sl = pl.ds(col, LANES)
                rows[r, sl] = rows[r, sl] * SCALE

            plsc.parallel_loop(0, CHUNK * D_MODEL // LANES, 1, unroll=8)(
                scale_body)
            pltpu.sync_copy(rows, out_hbm.at[pl.ds(base + g * CHUNK, CHUNK)])

        gather_start(0, 0)

        def pair_body(h, carry):
            for b in range(2):
                g = 2 * h + b

                @pl.when(g + 1 < NCHUNKS)
                def _start_next():
                    gather_start(g + 1, 1 - b)

                pltpu.make_async_copy(
                    table_hbm.at[idx_v.at[pl.ds(g * CHUNK, CHUNK)]],
                    bufs[b], sems[b]).wait()
                scale_store(g, b)
            return carry

        lax.fori_loop(0, NCHUNKS // 2, pair_body, 0)

    return emb_kernel(idx_flat, table)


def kernel(x, table):
    idx_flat = x.T.reshape(-1).astype(jnp.int32)
    out = _embed(idx_flat, table)
    return out.reshape(200, 4096, D_MODEL).transpose(1, 0, 2)
